# baseline (speedup 1.0000x reference)
import functools

import jax
import jax.numpy as jnp
from jax import lax
from jax.experimental import pallas as pl
from jax.experimental.pallas import tpu as pltpu
from jax.experimental.pallas import tpu_sc as plsc

N = 10000
D = 128
NC = 2
NS = 16
CH = 128
CPW = 160
NPASS = 2
NRANGE = NPASS * NC
RR = 2504
HROWS = 2560
RPS = HROWS // NS
TRASH = RR + 16
PAD_ROW = N + 500
NBUF = 4
DEGW = 128


def _remap_chunk(dst_v, j, lo, idx2_v, b):
    for k in range(CH // 16):
        d = dst_v[j, pl.ds(k * 16, 16)]
        dl = d - lo
        ok = (dl >= 0) & (dl < RR)
        idx2_v[b, pl.ds(k * 16, 16)] = jnp.where(ok, dl, TRASH)


def _sc_degree_body(dst_hbm, ones_hbm, zeros_hbm, deg_out, dst_v, ones_v,
                    idx2_v, deg_sh, *sems):
    cid = lax.axis_index("c")
    sid = lax.axis_index("s")
    base = sid * RPS

    pltpu.sync_copy(dst_hbm.at[sid], dst_v)
    pltpu.sync_copy(ones_hbm, ones_v)

    for p in range(NPASS):
        rg = p * NC + cid
        lo = rg * RR
        pltpu.sync_copy(zeros_hbm, deg_sh.at[pl.ds(base, RPS)])
        plsc.subcore_barrier()

        def body(i, carry):
            for b in range(NBUF):
                j = i * NBUF + b
                _remap_chunk(dst_v, j, lo, idx2_v, b)
                pltpu.async_copy(ones_v, deg_sh.at[idx2_v.at[b]], sems[b],
                                 add=True)
            for b in range(NBUF):
                j = i * NBUF + b
                pltpu.make_async_copy(ones_v, deg_sh.at[idx2_v.at[b]],
                                      sems[b]).wait()
            return carry

        lax.fori_loop(0, CPW // NBUF, body, 0)
        plsc.subcore_barrier()
        pltpu.sync_copy(deg_sh.at[pl.ds(base, RPS)],
                        deg_out.at[rg, pl.ds(base, RPS)])
        if p + 1 < NPASS:
            plsc.subcore_barrier()


def _sc_edge_body(g_hbm, src_hbm, dst_hbm, zeros_hbm, acc_out,
                  src_v, dst_v, idx2_v, rows_v, acc_sh, *sems):
    gsems = sems[:NBUF]
    ssems = sems[NBUF:]
    cid = lax.axis_index("c")
    sid = lax.axis_index("s")
    base = sid * RPS

    pltpu.sync_copy(src_hbm.at[sid], src_v)
    pltpu.sync_copy(dst_hbm.at[sid], dst_v)

    for p in range(NPASS):
        rg = p * NC + cid
        lo = rg * RR
        pltpu.sync_copy(zeros_hbm, acc_sh.at[pl.ds(base, RPS)])
        plsc.subcore_barrier()

        for b in range(NBUF):
            pltpu.async_copy(g_hbm.at[src_v.at[b]], rows_v.at[b], gsems[b])

        def step(i, carry):
            for b in range(NBUF):
                j = i * NBUF + b
                _remap_chunk(dst_v, j, lo, idx2_v, b)
                pltpu.make_async_copy(g_hbm.at[src_v.at[j]], rows_v.at[b],
                                      gsems[b]).wait()
                pltpu.async_copy(rows_v.at[b], acc_sh.at[idx2_v.at[b]],
                                 ssems[b], add=True)
            for b in range(NBUF):
                j = i * NBUF + b
                pltpu.make_async_copy(rows_v.at[b], acc_sh.at[idx2_v.at[b]],
                                      ssems[b]).wait()

                @pl.when(j + NBUF < CPW)
                def _():
                    pltpu.async_copy(g_hbm.at[src_v.at[j + NBUF]],
                                     rows_v.at[b], gsems[b])
            return carry

        lax.fori_loop(0, CPW // NBUF, step, 0)
        plsc.subcore_barrier()
        pltpu.sync_copy(acc_sh.at[pl.ds(base, RPS)],
                        acc_out.at[rg, pl.ds(base, RPS)])
        if p + 1 < NPASS:
            plsc.subcore_barrier()


@functools.cache
def _get_sc_kernels():
    mesh = plsc.VectorSubcoreMesh(core_axis_name="c", subcore_axis_name="s",
                                  num_cores=NC)
    sc_degree = functools.partial(
        pl.kernel,
        mesh=mesh,
        out_type=jax.ShapeDtypeStruct((NRANGE, HROWS, DEGW), jnp.float32),
        scratch_types=[
            pltpu.VMEM((CPW, CH), jnp.int32),
            pltpu.VMEM((CH, DEGW), jnp.float32),
            pltpu.VMEM((NBUF, CH), jnp.int32),
            pltpu.VMEM_SHARED((HROWS, DEGW), jnp.float32),
        ] + [pltpu.SemaphoreType.DMA] * NBUF,
    )(_sc_degree_body)
    sc_edge = functools.partial(
        pl.kernel,
        mesh=mesh,
        out_type=jax.ShapeDtypeStruct((NRANGE, HROWS, D), jnp.float32),
        scratch_types=[
            pltpu.VMEM((CPW, CH), jnp.int32),
            pltpu.VMEM((CPW, CH), jnp.int32),
            pltpu.VMEM((NBUF, CH), jnp.int32),
            pltpu.VMEM((NBUF, CH, D), jnp.float32),
            pltpu.VMEM_SHARED((HROWS, D), jnp.float32),
        ] + [pltpu.SemaphoreType.DMA] * (2 * NBUF),
    )(_sc_edge_body)
    return sc_degree, sc_edge


_GRID = NRANGE


def _dis(deg_ref):
    cnt = deg_ref[0, :, 0]
    return lax.rsqrt(cnt + 1.0)


def _tc_first_body(deg_ref, x_ref, w_ref, g_ref):
    dis = _dis(deg_ref)
    h = jnp.dot(x_ref[...], w_ref[...], preferred_element_type=jnp.float32)
    g_ref[...] = h * dis[:, None]


def _tc_mid_body(deg_ref, p_ref, gp_ref, b_ref, w_ref, g_ref):
    dis = _dis(deg_ref)
    s = p_ref[0] + gp_ref[...]
    out = jnp.maximum(s * dis[:, None] + b_ref[...], 0.0)
    g_ref[...] = jnp.dot(out, w_ref[...],
                         preferred_element_type=jnp.float32) * dis[:, None]


def _tc_head_body(deg_ref, p_ref, gp_ref, b_ref, m1_ref, mb1_ref, m2_ref,
                  mb2_ref, logits_ref, probs_ref, emb_ref):
    dis = _dis(deg_ref)
    s = p_ref[0] + gp_ref[...]
    emb = jnp.maximum(s * dis[:, None] + b_ref[...], 0.0)
    emb_ref[...] = emb
    z = jnp.dot(emb, m1_ref[...], preferred_element_type=jnp.float32)
    z = z + mb1_ref[...]
    z = jnp.where(z > 0.0, z, jnp.exp(jnp.minimum(z, 0.0)) - 1.0)
    logits = jnp.dot(z, m2_ref[...], preferred_element_type=jnp.float32)
    logits = logits + mb2_ref[...]
    logits_ref[...] = logits
    m = jnp.max(logits, axis=-1, keepdims=True)
    e = jnp.exp(logits - m)
    probs_ref[...] = e / jnp.sum(e, axis=-1, keepdims=True)


def _rb_spec(w):
    return pl.BlockSpec((RR, w), lambda i: (i, 0))


def _full_spec(h, w):
    return pl.BlockSpec((h, w), lambda i: (0, 0))


_deg_spec = pl.BlockSpec((1, RR, DEGW), lambda i: (i, 0, 0))
_p_spec = pl.BlockSpec((1, RR, D), lambda i: (i, 0, 0))

_tc_first = pl.pallas_call(
    _tc_first_body,
    grid=(_GRID,),
    in_specs=[_deg_spec, _rb_spec(D), _full_spec(D, D)],
    out_specs=_rb_spec(D),
    out_shape=jax.ShapeDtypeStruct((N, D), jnp.float32),
)

_tc_mid = pl.pallas_call(
    _tc_mid_body,
    grid=(_GRID,),
    in_specs=[_deg_spec, _p_spec, _rb_spec(D), _full_spec(1, D),
              _full_spec(D, D)],
    out_specs=_rb_spec(D),
    out_shape=jax.ShapeDtypeStruct((N, D), jnp.float32),
)

_NCLS = 16

_tc_head = pl.pallas_call(
    _tc_head_body,
    grid=(_GRID,),
    in_specs=[_deg_spec, _p_spec, _rb_spec(D), _full_spec(1, D),
              _full_spec(D, D), _full_spec(1, D), _full_spec(D, _NCLS),
              _full_spec(1, _NCLS)],
    out_specs=[_rb_spec(_NCLS), _rb_spec(_NCLS), _rb_spec(D)],
    out_shape=[
        jax.ShapeDtypeStruct((N, _NCLS), jnp.float32),
        jax.ShapeDtypeStruct((N, _NCLS), jnp.float32),
        jax.ShapeDtypeStruct((N, D), jnp.float32),
    ],
)


def kernel(x, edge_index, W1, b1, W2, b2, W3, b3, M1, mb1, M2, mb2):
    src = edge_index[0].astype(jnp.int32)
    dst = edge_index[1].astype(jnp.int32)
    e = src.shape[0]
    epad = NS * CPW * CH
    src_p = jnp.concatenate(
        [src, jnp.zeros((epad - e,), jnp.int32)]).reshape(NS, CPW, CH)
    dst_p = jnp.concatenate(
        [dst, jnp.full((epad - e,), PAD_ROW, jnp.int32)]).reshape(NS, CPW, CH)

    ones_d = jnp.ones((CH, DEGW), jnp.float32)
    zeros_d = jnp.zeros((RPS, DEGW), jnp.float32)
    zeros_s = jnp.zeros((RPS, D), jnp.float32)

    sc_degree, sc_edge = _get_sc_kernels()
    deg = sc_degree(dst_p, ones_d, zeros_d)

    g1 = _tc_first(deg, x, W1)

    ws = jnp.stack([W2, W3, W3])
    bs = jnp.stack([b1.reshape(1, D), b2.reshape(1, D), b2.reshape(1, D)])

    def layer(carry, wb):
        g, _, _ = carry
        w, b = wb
        p = sc_edge(g, src_p, dst_p, zeros_s)
        g_next = _tc_mid(deg, p, g, b, w)
        return (g_next, p, g), None

    dummy_p = jnp.zeros((NRANGE, HROWS, D), jnp.float32)
    (_, p3, g3), _ = lax.scan(layer, (g1, dummy_p, g1), (ws, bs))

    logits, probs, emb = _tc_head(deg, p3, g3, b3.reshape(1, D), M1,
                                  mb1.reshape(1, D), M2,
                                  mb2.reshape(1, _NCLS))
    return (logits, probs, emb)


# speedup vs baseline: 3.9231x; 3.9231x over previous
import functools

import jax
import jax.numpy as jnp
from jax import lax
from jax.experimental import pallas as pl
from jax.experimental.pallas import tpu as pltpu
from jax.experimental.pallas import tpu_sc as plsc

N = 10000
D = 128
NC = 2
NS = 16
CH = 128
CPW = 160
NPASS = 2
NRANGE = NPASS * NC
RR = 2504
HROWS = 2560
RPS = HROWS // NS
TRASH = RR + 16
PAD_ROW = N + 500
NBUF = 4
DEGW = 128
NW = NC * NS
CPW2 = 80
FILL = 512
CAP = CPW2 * CH + FILL


def _sc_compact_body(src_hbm, dst_hbm, csrc_out, cdst_out, ncnt_out,
                     src_v, dst_v, csrc_v, cdst_v, ncnt_v):
    cid = lax.axis_index("c")
    sid = lax.axis_index("s")
    w = cid * NS + sid

    pltpu.sync_copy(src_hbm.at[w], src_v)
    pltpu.sync_copy(dst_hbm.at[w], dst_v)
    lanes = jnp.arange(16, dtype=jnp.int32)

    def _prefix(x, tmp_v):
        for sh in (1, 2, 4, 8):
            tmp_v[pl.ds(0, 16)] = x
            shifted = plsc.load_gather(tmp_v, [jnp.maximum(lanes - sh, 0)])
            x = x + jnp.where(lanes >= sh, shifted, 0)
        return x

    for rg in range(NRANGE):
        lo = rg * RR

        def chunk(j, cnt):
            for k in range(CH // 16):
                s = src_v[j, pl.ds(k * 16, 16)]
                d = dst_v[j, pl.ds(k * 16, 16)]
                dl = d - lo
                ok = (dl >= 0) & (dl < RR)
                pfx = _prefix(ok.astype(jnp.int32), ncnt_v)
                pos = jnp.where(ok, cnt + pfx - 1, CAP + lanes)
                plsc.store_scatter(csrc_v, [pos], s)
                plsc.store_scatter(cdst_v, [pos], dl)
                cnt = cnt + plsc.all_reduce_population_count(ok)
            return cnt

        cnt = lax.fori_loop(0, CPW2, chunk, jnp.zeros((16,), jnp.int32))
        zsrc = jnp.zeros((16,), jnp.int32)
        ztrash = jnp.full((16,), TRASH, jnp.int32)
        for f in range(FILL // 16):
            fpos = cnt + f * 16 + lanes
            plsc.store_scatter(csrc_v, [fpos], zsrc)
            plsc.store_scatter(cdst_v, [fpos], ztrash)
        ncnt_v[pl.ds(0, 16)] = cnt
        pltpu.sync_copy(csrc_v.at[pl.ds(0, CAP)], csrc_out.at[rg, w])
        pltpu.sync_copy(cdst_v.at[pl.ds(0, CAP)], cdst_out.at[rg, w])
        pltpu.sync_copy(ncnt_v, ncnt_out.at[rg, w])


def _load_half(ncnt_hbm, cdst_hbm, rg, w, ncnt_v, cdst_v):
    pltpu.sync_copy(ncnt_hbm.at[rg, w], ncnt_v)
    pltpu.sync_copy(cdst_hbm.at[rg, w], cdst_v)
    n = ncnt_v[pl.ds(0, 16)][0]
    return (n + CH - 1) // CH


def _stage_idx_chunk(cdst_v, j, idx2_v, b):
    for k in range(CH // 16):
        idx2_v[b, pl.ds(k * 16, 16)] = cdst_v[pl.ds(j * CH + k * 16, 16)]


def _sc_degree_body(cdst_hbm, ncnt_hbm, ones_hbm, zeros_hbm, deg_out,
                    cdst_v, ncnt_v, ones_v, idx2_v, deg_sh):
    cid = lax.axis_index("c")
    sid = lax.axis_index("s")
    base = sid * RPS

    pltpu.sync_copy(ones_hbm, ones_v)

    for p in range(NPASS):
        rg = p * NC + cid
        pltpu.sync_copy(zeros_hbm, deg_sh.at[pl.ds(base, RPS)])
        plsc.subcore_barrier()

        for h in range(NC):
            w = h * NS + sid
            nch = _load_half(ncnt_hbm, cdst_hbm, rg, w, ncnt_v, cdst_v)

            def body(j, carry):
                _stage_idx_chunk(cdst_v, j, idx2_v, 0)
                pltpu.sync_copy(ones_v, deg_sh.at[idx2_v.at[0]], add=True)
                return carry

            lax.fori_loop(0, nch, body, 0)

        plsc.subcore_barrier()
        pltpu.sync_copy(deg_sh.at[pl.ds(base, RPS)],
                        deg_out.at[rg, pl.ds(base, RPS)])
        if p + 1 < NPASS:
            plsc.subcore_barrier()


def _sc_edge_body(g_hbm, csrc_hbm, cdst_hbm, ncnt_hbm, zeros_hbm, acc_out,
                  csrc_v, cdst_v, ncnt_v, idx2_v, rows_v, acc_sh, *sems):
    cid = lax.axis_index("c")
    sid = lax.axis_index("s")
    base = sid * RPS

    for p in range(NPASS):
        rg = p * NC + cid
        pltpu.sync_copy(zeros_hbm, acc_sh.at[pl.ds(base, RPS)])
        plsc.subcore_barrier()

        for h in range(NC):
            w = h * NS + sid
            nch = _load_half(ncnt_hbm, cdst_hbm, rg, w, ncnt_v, cdst_v)
            pltpu.sync_copy(csrc_hbm.at[rg, w], csrc_v)

            for b in range(NBUF):
                @pl.when(b < nch)
                def _():
                    pltpu.async_copy(g_hbm.at[csrc_v.at[pl.ds(b * CH, CH)]],
                                     rows_v.at[b], sems[b])

            def step(i, carry):
                for b in range(NBUF):
                    j = i * NBUF + b

                    @pl.when(j < nch)
                    def _():
                        _stage_idx_chunk(cdst_v, j, idx2_v, b)
                        pltpu.make_async_copy(
                            g_hbm.at[csrc_v.at[pl.ds(j * CH, CH)]],
                            rows_v.at[b], sems[b]).wait()
                        pltpu.sync_copy(rows_v.at[b],
                                        acc_sh.at[idx2_v.at[b]], add=True)

                        @pl.when(j + NBUF < nch)
                        def _():
                            pltpu.async_copy(
                                g_hbm.at[csrc_v.at[pl.ds((j + NBUF) * CH,
                                                         CH)]],
                                rows_v.at[b], sems[b])
                return carry

            lax.fori_loop(0, (nch + NBUF - 1) // NBUF, step, 0)

        plsc.subcore_barrier()
        pltpu.sync_copy(acc_sh.at[pl.ds(base, RPS)],
                        acc_out.at[rg, pl.ds(base, RPS)])
        if p + 1 < NPASS:
            plsc.subcore_barrier()


@functools.cache
def _get_sc_kernels():
    mesh = plsc.VectorSubcoreMesh(core_axis_name="c", subcore_axis_name="s",
                                  num_cores=NC)
    sc_compact = functools.partial(
        pl.kernel,
        mesh=mesh,
        out_type=[
            jax.ShapeDtypeStruct((NRANGE, NW, CAP), jnp.int32),
            jax.ShapeDtypeStruct((NRANGE, NW, CAP), jnp.int32),
            jax.ShapeDtypeStruct((NRANGE, NW, 16), jnp.int32),
        ],
        scratch_types=[
            pltpu.VMEM((CPW2, CH), jnp.int32),
            pltpu.VMEM((CPW2, CH), jnp.int32),
            pltpu.VMEM((CAP + 16,), jnp.int32),
            pltpu.VMEM((CAP + 16,), jnp.int32),
            pltpu.VMEM((16,), jnp.int32),
        ],
        compiler_params=pltpu.CompilerParams(needs_layout_passes=False),
    )(_sc_compact_body)
    sc_degree = functools.partial(
        pl.kernel,
        mesh=mesh,
        out_type=jax.ShapeDtypeStruct((NRANGE, HROWS, DEGW), jnp.float32),
        scratch_types=[
            pltpu.VMEM((CAP,), jnp.int32),
            pltpu.VMEM((16,), jnp.int32),
            pltpu.VMEM((CH, DEGW), jnp.float32),
            pltpu.VMEM((1, CH), jnp.int32),
            pltpu.VMEM_SHARED((HROWS, DEGW), jnp.float32),
        ],
    )(_sc_degree_body)
    sc_edge = functools.partial(
        pl.kernel,
        mesh=mesh,
        out_type=jax.ShapeDtypeStruct((NRANGE, HROWS, D), jnp.float32),
        scratch_types=[
            pltpu.VMEM((CAP,), jnp.int32),
            pltpu.VMEM((CAP,), jnp.int32),
            pltpu.VMEM((16,), jnp.int32),
            pltpu.VMEM((NBUF, CH), jnp.int32),
            pltpu.VMEM((NBUF, CH, D), jnp.float32),
            pltpu.VMEM_SHARED((HROWS, D), jnp.float32),
        ] + [pltpu.SemaphoreType.DMA] * NBUF,
    )(_sc_edge_body)
    return sc_compact, sc_degree, sc_edge


_GRID = NRANGE


def _dis(deg_ref):
    cnt = deg_ref[0, :, 0]
    return lax.rsqrt(cnt + 1.0)


def _tc_first_body(deg_ref, x_ref, w_ref, g_ref):
    dis = _dis(deg_ref)
    h = jnp.dot(x_ref[...], w_ref[...], preferred_element_type=jnp.float32)
    g_ref[...] = h * dis[:, None]


def _tc_mid_body(deg_ref, p_ref, gp_ref, b_ref, w_ref, g_ref):
    dis = _dis(deg_ref)
    s = p_ref[0] + gp_ref[...]
    out = jnp.maximum(s * dis[:, None] + b_ref[...], 0.0)
    g_ref[...] = jnp.dot(out, w_ref[...],
                         preferred_element_type=jnp.float32) * dis[:, None]


def _tc_head_body(deg_ref, p_ref, gp_ref, b_ref, m1_ref, mb1_ref, m2_ref,
                  mb2_ref, logits_ref, probs_ref, emb_ref):
    dis = _dis(deg_ref)
    s = p_ref[0] + gp_ref[...]
    emb = jnp.maximum(s * dis[:, None] + b_ref[...], 0.0)
    emb_ref[...] = emb
    z = jnp.dot(emb, m1_ref[...], preferred_element_type=jnp.float32)
    z = z + mb1_ref[...]
    z = jnp.where(z > 0.0, z, jnp.exp(jnp.minimum(z, 0.0)) - 1.0)
    logits = jnp.dot(z, m2_ref[...], preferred_element_type=jnp.float32)
    logits = logits + mb2_ref[...]
    logits_ref[...] = logits
    m = jnp.max(logits, axis=-1, keepdims=True)
    e = jnp.exp(logits - m)
    probs_ref[...] = e / jnp.sum(e, axis=-1, keepdims=True)


def _rb_spec(w):
    return pl.BlockSpec((RR, w), lambda i: (i, 0))


def _full_spec(h, w):
    return pl.BlockSpec((h, w), lambda i: (0, 0))


_deg_spec = pl.BlockSpec((1, RR, DEGW), lambda i: (i, 0, 0))
_p_spec = pl.BlockSpec((1, RR, D), lambda i: (i, 0, 0))

_tc_first = pl.pallas_call(
    _tc_first_body,
    grid=(_GRID,),
    in_specs=[_deg_spec, _rb_spec(D), _full_spec(D, D)],
    out_specs=_rb_spec(D),
    out_shape=jax.ShapeDtypeStruct((N, D), jnp.float32),
)

_tc_mid = pl.pallas_call(
    _tc_mid_body,
    grid=(_GRID,),
    in_specs=[_deg_spec, _p_spec, _rb_spec(D), _full_spec(1, D),
              _full_spec(D, D)],
    out_specs=_rb_spec(D),
    out_shape=jax.ShapeDtypeStruct((N, D), jnp.float32),
)

_NCLS = 16

_tc_head = pl.pallas_call(
    _tc_head_body,
    grid=(_GRID,),
    in_specs=[_deg_spec, _p_spec, _rb_spec(D), _full_spec(1, D),
              _full_spec(D, D), _full_spec(1, D), _full_spec(D, _NCLS),
              _full_spec(1, _NCLS)],
    out_specs=[_rb_spec(_NCLS), _rb_spec(_NCLS), _rb_spec(D)],
    out_shape=[
        jax.ShapeDtypeStruct((N, _NCLS), jnp.float32),
        jax.ShapeDtypeStruct((N, _NCLS), jnp.float32),
        jax.ShapeDtypeStruct((N, D), jnp.float32),
    ],
)


def kernel(x, edge_index, W1, b1, W2, b2, W3, b3, M1, mb1, M2, mb2):
    src = edge_index[0].astype(jnp.int32)
    dst = edge_index[1].astype(jnp.int32)
    e = src.shape[0]
    epad = NW * CPW2 * CH
    src_p = jnp.concatenate(
        [src, jnp.zeros((epad - e,), jnp.int32)]).reshape(NW, CPW2, CH)
    dst_p = jnp.concatenate(
        [dst, jnp.full((epad - e,), PAD_ROW, jnp.int32)]).reshape(NW, CPW2, CH)

    ones_d = jnp.ones((CH, DEGW), jnp.float32)
    zeros_d = jnp.zeros((RPS, DEGW), jnp.float32)
    zeros_s = jnp.zeros((RPS, D), jnp.float32)

    sc_compact, sc_degree, sc_edge = _get_sc_kernels()
    csrc, cdst, ncnt = sc_compact(src_p, dst_p)
    deg = sc_degree(cdst, ncnt, ones_d, zeros_d)

    g1 = _tc_first(deg, x, W1)

    ws = jnp.stack([W2, W3, W3])
    bs = jnp.stack([b1.reshape(1, D), b2.reshape(1, D), b2.reshape(1, D)])

    def layer(carry, wb):
        g, _, _ = carry
        w, b = wb
        p = sc_edge(g, csrc, cdst, ncnt, zeros_s)
        g_next = _tc_mid(deg, p, g, b, w)
        return (g_next, p, g), None

    dummy_p = jnp.zeros((NRANGE, HROWS, D), jnp.float32)
    (_, p3, g3), _ = lax.scan(layer, (g1, dummy_p, g1), (ws, bs))

    logits, probs, emb = _tc_head(deg, p3, g3, b3.reshape(1, D), M1,
                                  mb1.reshape(1, D), M2,
                                  mb2.reshape(1, _NCLS))
    return (logits, probs, emb)
